# Initial kernel scaffold; baseline (speedup 1.0000x reference)
import jax
import jax.numpy as jnp
from jax.experimental import pallas as pl


def kernel(x, lengths, table, W1, b1, Wn, bn, W3, b3):
    raise NotImplementedError("write your pallas kernel here")


# speedup vs baseline: 8.8763x; 8.8763x over previous
import functools

import jax
import jax.numpy as jnp
from jax import lax
from jax.experimental import pallas as pl
from jax.experimental.pallas import tpu as pltpu
from jax.experimental.pallas import tpu_sc as plsc

_B, _L, _V, _D, _OUT = 4096, 200, 100000, 64, 10
_CH = _L // 2
_LANES = 16
_NVR = _D // _LANES

_info = plsc.get_sparse_core_info()
_NC, _NS = _info.num_cores, _info.num_subcores
_NW = _NC * _NS
_EPW = _B // _NW


def _sc_pool_body(x_hbm, table_hbm, out_hbm, idx_v, rows_v, outbuf, sem0, sem1):
    wid = lax.axis_index("s") * _NC + lax.axis_index("c")
    base = wid * _EPW
    pltpu.sync_copy(x_hbm.at[pl.ds(base, _EPW)], idx_v)

    def elem(e, _):
        c0 = pltpu.async_copy(table_hbm.at[idx_v.at[e, 0]],
                              rows_v.at[pl.ds(0, _CH)], sem0)
        c1 = pltpu.async_copy(table_hbm.at[idx_v.at[e, 1]],
                              rows_v.at[pl.ds(_CH, _CH)], sem1)
        c0.wait()
        c1.wait()

        init = []
        for q in range(_NVR):
            v0 = rows_v[0, pl.ds(q * _LANES, _LANES)]
            init += [v0, v0]

        def row(j, accs):
            new = []
            for q in range(_NVR):
                v = rows_v[j, pl.ds(q * _LANES, _LANES)]
                new.append(accs[2 * q] + v)
                new.append(jnp.maximum(accs[2 * q + 1], v))
            return tuple(new)

        accs = lax.fori_loop(1, _L, row, tuple(init))
        for q in range(_NVR):
            outbuf[e, pl.ds(q * _LANES, _LANES)] = accs[2 * q]
            outbuf[e, pl.ds(_D + q * _LANES, _LANES)] = accs[2 * q + 1]
        return 0

    lax.fori_loop(0, _EPW, elem, 0)
    pltpu.sync_copy(outbuf, out_hbm.at[pl.ds(base, _EPW)])


_sc_pool = functools.partial(
    pl.kernel,
    mesh=plsc.VectorSubcoreMesh(core_axis_name="c", subcore_axis_name="s"),
    out_type=jax.ShapeDtypeStruct((_B, 2 * _D), jnp.float32),
    scratch_types=[
        pltpu.VMEM((_EPW, 2, _CH), jnp.int32),
        pltpu.VMEM((_L, _D), jnp.float32),
        pltpu.VMEM((_EPW, 2 * _D), jnp.float32),
        pltpu.SemaphoreType.DMA,
        pltpu.SemaphoreType.DMA,
    ],
    compiler_params=pltpu.CompilerParams(use_tc_tiling_on_sc=False),
)(_sc_pool_body)


def _tc_head_body(pooled_ref, len_ref, wn_ref, bn_ref, w3_ref, b3_ref, out_ref):
    pooled = pooled_ref[...]
    lens = len_ref[...]
    inv2 = 1.0 / (lens * lens)
    mean = pooled[:, :_D] * inv2
    rep = jnp.concatenate([mean, pooled[:, _D:]], axis=1)
    h = lax.dot_general(rep, wn_ref[...], (((1,), (1,)), ((), ())),
                        preferred_element_type=jnp.float32)
    h = jnp.maximum(h + bn_ref[...], 0.0)
    out = lax.dot_general(h, w3_ref[...], (((1,), (1,)), ((), ())),
                          preferred_element_type=jnp.float32)
    out_ref[...] = out + b3_ref[...]


def kernel(x, lengths, table, W1, b1, Wn, bn, W3, b3):
    x3 = x.astype(jnp.int32).reshape(_B, 2, _CH)
    pooled = _sc_pool(x3, table)
    lens = lengths.astype(jnp.float32).reshape(_B, 1)
    out = pl.pallas_call(
        _tc_head_body,
        out_shape=jax.ShapeDtypeStruct((_B, _OUT), jnp.float32),
    )(pooled, lens, Wn, bn.reshape(1, -1), W3, b3.reshape(1, -1))
    return out


# speedup vs baseline: 13.4014x; 1.5098x over previous
import functools

import jax
import jax.numpy as jnp
from jax import lax
from jax.experimental import pallas as pl
from jax.experimental.pallas import tpu as pltpu
from jax.experimental.pallas import tpu_sc as plsc

_B, _L, _V, _D, _OUT = 4096, 200, 100000, 64, 10
_CH = _L // 2
_LANES = 16
_NVR = _D // _LANES

_info = plsc.get_sparse_core_info()
_NC, _NS = _info.num_cores, _info.num_subcores
_NW = _NC * _NS
_EPW = _B // _NW


_UNROLL = 8


def _sc_pool_body(x_hbm, table_hbm, out_hbm, idx_v, rows0, rows1, outbuf,
                  sem0, sem1):
    wid = lax.axis_index("s") * _NC + lax.axis_index("c")
    base = wid * _EPW
    pltpu.sync_copy(x_hbm.at[pl.ds(base, _EPW)], idx_v)

    bufs = ((rows0, sem0), (rows1, sem1))

    def issue(e, rows, sem):
        pltpu.async_copy(table_hbm.at[idx_v.at[e, 0]],
                         rows.at[pl.ds(0, _CH)], sem)
        pltpu.async_copy(table_hbm.at[idx_v.at[e, 1]],
                         rows.at[pl.ds(_CH, _CH)], sem)

    def drain(rows, sem):
        pltpu.make_async_copy(table_hbm.at[idx_v.at[0, 0]],
                              rows.at[pl.ds(0, _CH)], sem).wait()
        pltpu.make_async_copy(table_hbm.at[idx_v.at[0, 1]],
                              rows.at[pl.ds(_CH, _CH)], sem).wait()

    def compute(e, rows):
        zero = jnp.zeros((_LANES,), jnp.float32)
        ninf = jnp.full((_LANES,), -jnp.inf, jnp.float32)
        init = (zero,) * _NVR + (ninf,) * _NVR

        def rowstep(j0, accs):
            s = list(accs[:_NVR])
            m = list(accs[_NVR:])
            for u in range(_UNROLL):
                j = j0 * _UNROLL + u
                for q in range(_NVR):
                    v = rows[j, pl.ds(q * _LANES, _LANES)]
                    s[q] = s[q] + v
                    m[q] = jnp.maximum(m[q], v)
            return tuple(s) + tuple(m)

        accs = lax.fori_loop(0, _L // _UNROLL, rowstep, init)
        for q in range(_NVR):
            outbuf[e, pl.ds(q * _LANES, _LANES)] = accs[q]
            outbuf[e, pl.ds(_D + q * _LANES, _LANES)] = accs[_NVR + q]

    issue(0, rows0, sem0)
    issue(1, rows1, sem1)

    def body(e2, _):
        e = 2 * e2
        for b, (rows, sem) in enumerate(bufs):
            eb = e + b
            drain(rows, sem)
            compute(eb, rows)

            @pl.when(eb + 2 < _EPW)
            def _():
                issue(eb + 2, rows, sem)
        return 0

    lax.fori_loop(0, _EPW // 2, body, 0)
    pltpu.sync_copy(outbuf, out_hbm.at[pl.ds(base, _EPW)])


_sc_pool = functools.partial(
    pl.kernel,
    mesh=plsc.VectorSubcoreMesh(core_axis_name="c", subcore_axis_name="s"),
    out_type=jax.ShapeDtypeStruct((_B, 2 * _D), jnp.float32),
    scratch_types=[
        pltpu.VMEM((_EPW, 2, _CH), jnp.int32),
        pltpu.VMEM((_L, _D), jnp.float32),
        pltpu.VMEM((_L, _D), jnp.float32),
        pltpu.VMEM((_EPW, 2 * _D), jnp.float32),
        pltpu.SemaphoreType.DMA,
        pltpu.SemaphoreType.DMA,
    ],
    compiler_params=pltpu.CompilerParams(use_tc_tiling_on_sc=False),
)(_sc_pool_body)


def _tc_head_body(pooled_ref, len_ref, wn_ref, bn_ref, w3_ref, b3_ref, out_ref):
    pooled = pooled_ref[...]
    lens = len_ref[...]
    inv2 = 1.0 / (lens * lens)
    mean = pooled[:, :_D] * inv2
    rep = jnp.concatenate([mean, pooled[:, _D:]], axis=1)
    h = lax.dot_general(rep, wn_ref[...], (((1,), (1,)), ((), ())),
                        preferred_element_type=jnp.float32)
    h = jnp.maximum(h + bn_ref[...], 0.0)
    out = lax.dot_general(h, w3_ref[...], (((1,), (1,)), ((), ())),
                          preferred_element_type=jnp.float32)
    out_ref[...] = out + b3_ref[...]


def kernel(x, lengths, table, W1, b1, Wn, bn, W3, b3):
    x3 = x.astype(jnp.int32).reshape(_B, 2, _CH)
    pooled = _sc_pool(x3, table)
    lens = lengths.astype(jnp.float32).reshape(_B, 1)
    out = pl.pallas_call(
        _tc_head_body,
        out_shape=jax.ShapeDtypeStruct((_B, _OUT), jnp.float32),
    )(pooled, lens, Wn, bn.reshape(1, -1), W3, b3.reshape(1, -1))
    return out


# speedup vs baseline: 17.8074x; 1.3288x over previous
import functools

import jax
import jax.numpy as jnp
from jax import lax
from jax.experimental import pallas as pl
from jax.experimental.pallas import tpu as pltpu
from jax.experimental.pallas import tpu_sc as plsc

_B, _L, _V, _D, _OUT = 4096, 200, 100000, 64, 10
_CHA = 128
_CHB = _L - _CHA
_LANES = 16
_NVR = _D // _LANES

_info = plsc.get_sparse_core_info()
_NC, _NS = _info.num_cores, _info.num_subcores
_NW = _NC * _NS
_EPW = _B // _NW


_UNROLL = 8
_NBUF = 4


def _sc_pool_body(x_hbm, table_hbm, out_hbm, idx_v,
                  rows0, rows1, rows2, rows3, outbuf,
                  sem0, sem1, sem2, sem3):
    wid = lax.axis_index("s") * _NC + lax.axis_index("c")
    base = wid * _EPW
    pltpu.sync_copy(x_hbm.at[pl.ds(base, _EPW)], idx_v)

    bufs = ((rows0, sem0), (rows1, sem1), (rows2, sem2), (rows3, sem3))

    def issue(e, rows, sem):
        pltpu.async_copy(table_hbm.at[idx_v.at[e, pl.ds(0, _CHA)]],
                         rows.at[pl.ds(0, _CHA)], sem)
        pltpu.async_copy(table_hbm.at[idx_v.at[e, pl.ds(_CHA, _CHB)]],
                         rows.at[pl.ds(_CHA, _CHB)], sem)

    def drain(rows, sem):
        pltpu.make_async_copy(table_hbm.at[idx_v.at[0, pl.ds(0, _CHA)]],
                              rows.at[pl.ds(0, _CHA)], sem).wait()
        pltpu.make_async_copy(table_hbm.at[idx_v.at[0, pl.ds(_CHA, _CHB)]],
                              rows.at[pl.ds(_CHA, _CHB)], sem).wait()

    def compute(e, rows):
        zero = jnp.zeros((_LANES,), jnp.float32)
        ninf = jnp.full((_LANES,), -jnp.inf, jnp.float32)
        init = (zero,) * _NVR + (ninf,) * _NVR

        def rowstep(j0, accs):
            s = list(accs[:_NVR])
            m = list(accs[_NVR:])
            for u in range(_UNROLL):
                j = j0 * _UNROLL + u
                for q in range(_NVR):
                    v = rows[j, pl.ds(q * _LANES, _LANES)]
                    s[q] = s[q] + v
                    m[q] = jnp.maximum(m[q], v)
            return tuple(s) + tuple(m)

        accs = lax.fori_loop(0, _L // _UNROLL, rowstep, init)
        for q in range(_NVR):
            outbuf[e, pl.ds(q * _LANES, _LANES)] = accs[q]
            outbuf[e, pl.ds(_D + q * _LANES, _LANES)] = accs[_NVR + q]

    for b, (rows, sem) in enumerate(bufs):
        issue(b, rows, sem)

    def body(eg, _):
        e = _NBUF * eg
        for b, (rows, sem) in enumerate(bufs):
            eb = e + b
            drain(rows, sem)
            compute(eb, rows)

            @pl.when(eb + _NBUF < _EPW)
            def _():
                issue(eb + _NBUF, rows, sem)
        return 0

    lax.fori_loop(0, _EPW // _NBUF, body, 0)
    pltpu.sync_copy(outbuf, out_hbm.at[pl.ds(base, _EPW)])


_sc_pool = functools.partial(
    pl.kernel,
    mesh=plsc.VectorSubcoreMesh(core_axis_name="c", subcore_axis_name="s"),
    out_type=jax.ShapeDtypeStruct((_B, 2 * _D), jnp.float32),
    scratch_types=[
        pltpu.VMEM((_EPW, _L), jnp.int32),
        pltpu.VMEM((_L, _D), jnp.float32),
        pltpu.VMEM((_L, _D), jnp.float32),
        pltpu.VMEM((_L, _D), jnp.float32),
        pltpu.VMEM((_L, _D), jnp.float32),
        pltpu.VMEM((_EPW, 2 * _D), jnp.float32),
        pltpu.SemaphoreType.DMA,
        pltpu.SemaphoreType.DMA,
        pltpu.SemaphoreType.DMA,
        pltpu.SemaphoreType.DMA,
    ],
    compiler_params=pltpu.CompilerParams(use_tc_tiling_on_sc=False),
)(_sc_pool_body)


def _tc_head_body(pooled_ref, len_ref, wn_ref, bn_ref, w3_ref, b3_ref, out_ref):
    pooled = pooled_ref[...]
    lens = len_ref[...]
    inv2 = 1.0 / (lens * lens)
    mean = pooled[:, :_D] * inv2
    rep = jnp.concatenate([mean, pooled[:, _D:]], axis=1)
    h = lax.dot_general(rep, wn_ref[...], (((1,), (1,)), ((), ())),
                        preferred_element_type=jnp.float32)
    h = jnp.maximum(h + bn_ref[...], 0.0)
    out = lax.dot_general(h, w3_ref[...], (((1,), (1,)), ((), ())),
                          preferred_element_type=jnp.float32)
    out_ref[...] = out + b3_ref[...]


def kernel(x, lengths, table, W1, b1, Wn, bn, W3, b3):
    pooled = _sc_pool(x.astype(jnp.int32), table)
    lens = lengths.astype(jnp.float32).reshape(_B, 1)
    out = pl.pallas_call(
        _tc_head_body,
        out_shape=jax.ShapeDtypeStruct((_B, _OUT), jnp.float32),
    )(pooled, lens, Wn, bn.reshape(1, -1), W3, b3.reshape(1, -1))
    return out
